# baseline (speedup 1.0000x reference)
import jax
import jax.numpy as jnp
from jax import lax
from jax.experimental import pallas as pl
from jax.experimental.pallas import tpu as pltpu
from jax.experimental.pallas import tpu_sc as plsc

_N = 10000
_E = 320000
_D = 128
_CHUNK = 64
_NC = 2
_NS = 16
_NW = _NC * _NS
_E_PAD = 327680
_EPW = _E_PAD // _NW
_NACC = 10112
_RPT = _NACC // _NS


_EPH = _EPW // 2
_CPH = _EPH // _CHUNK
_G = 4
_DEPTH = 2
_NGRP = _CPH // _G


def _spmm_body(z_hbm, src_hbm, dst_hbm, zeros_hbm, out_hbm,
               src_v, dst_v, didx0, didx1, didx2, didx3,
               rows0, rows1, rows2, rows3, acc,
               gsem0, gsem1, gsem2, gsem3, ssem0, ssem1, ssem2, ssem3):
    rows = (rows0, rows1, rows2, rows3)
    didx = (didx0, didx1, didx2, didx3)
    gsem = (gsem0, gsem1, gsem2, gsem3)
    ssem = (ssem0, ssem1, ssem2, ssem3)
    c = lax.axis_index("c")
    s = lax.axis_index("s")
    wid = s * _NC + c

    def stage(h):
        pltpu.sync_copy(src_hbm.at[pl.ds(wid * _EPW + h * _EPH, _EPH)],
                        src_v)
        pltpu.sync_copy(dst_hbm.at[pl.ds(wid * _EPW + h * _EPH, _EPH)],
                        dst_v)

    pltpu.sync_copy(zeros_hbm, acc.at[pl.ds(s * _RPT, _RPT)])
    stage(0)
    plsc.subcore_barrier()

    def copy_didx(lch, b):
        for k in range(_CHUNK // 16):
            didx[b][pl.ds(k * 16, 16)] = dst_v[pl.ds(lch * _CHUNK + k * 16,
                                                     16)]

    def fire_gather(lch, b):
        pltpu.async_copy(z_hbm.at[src_v.at[pl.ds(lch * _CHUNK, _CHUNK)]],
                         rows[b], gsem[b])

    def wait_gather(lch, b):
        pltpu.make_async_copy(z_hbm.at[src_v.at[pl.ds(lch * _CHUNK,
                                                      _CHUNK)]],
                              rows[b], gsem[b]).wait()

    def fire_scatter(b):
        pltpu.async_copy(rows[b], acc.at[didx[b]], ssem[b], add=True)

    def wait_scatter(b):
        pltpu.make_async_copy(rows[b], acc.at[didx[b]], ssem[b]).wait()

    def run_half(first):
        for j in range(_G):
            if not first:
                wait_scatter(j)
            copy_didx(j, j)
            fire_gather(j, j)
            if j >= _DEPTH:
                wait_gather(j - _DEPTH, j - _DEPTH)
                fire_scatter(j - _DEPTH)

        @pl.loop(1, _NGRP)
        def _(g):
            for b in range(_G):
                k = g * _G + b
                wait_scatter(b)
                copy_didx(k, b)
                fire_gather(k, b)
                bd = (b + _G - _DEPTH) % _G
                wait_gather(k - _DEPTH, bd)
                fire_scatter(bd)

        for j in range(_CPH - _DEPTH, _CPH):
            bd = j % _G
            wait_gather(j, bd)
            fire_scatter(bd)

    run_half(True)
    stage(1)
    run_half(False)

    for b in range(_G):
        wait_scatter(b)

    plsc.subcore_barrier()
    pltpu.sync_copy(acc.at[pl.ds(s * _RPT, _RPT)],
                    out_hbm.at[c, pl.ds(s * _RPT, _RPT)])


_spmm = pl.kernel(
    _spmm_body,
    out_type=jax.ShapeDtypeStruct((_NC, _NACC, _D), jnp.float32),
    mesh=plsc.VectorSubcoreMesh(core_axis_name="c", subcore_axis_name="s",
                                num_cores=_NC, num_subcores=_NS),
    scratch_types=[
        pltpu.VMEM((_EPH,), jnp.int32),
        pltpu.VMEM((_EPH,), jnp.int32),
        pltpu.VMEM((_CHUNK,), jnp.int32),
        pltpu.VMEM((_CHUNK,), jnp.int32),
        pltpu.VMEM((_CHUNK,), jnp.int32),
        pltpu.VMEM((_CHUNK,), jnp.int32),
        pltpu.VMEM((_CHUNK, _D), jnp.float32),
        pltpu.VMEM((_CHUNK, _D), jnp.float32),
        pltpu.VMEM((_CHUNK, _D), jnp.float32),
        pltpu.VMEM((_CHUNK, _D), jnp.float32),
        pltpu.VMEM_SHARED((_NACC, _D), jnp.float32),
    ] + [pltpu.SemaphoreType.DMA] * (2 * _G),
)


def _mm_first_body(y_ref, w_ref, b_ref, o_ref):
    o_ref[...] = lax.dot_general(
        y_ref[...], w_ref[...], (((1,), (1,)), ((), ())),
        preferred_element_type=jnp.float32) + b_ref[...]


_mm_first = pl.pallas_call(
    _mm_first_body,
    out_shape=jax.ShapeDtypeStruct((_N, _D), jnp.float32),
)


def _mm_mid_body(p_ref, w_ref, b_ref, o_ref):
    x = jnp.maximum(p_ref[0, :_N] + p_ref[1, :_N], 0.0)
    o_ref[...] = lax.dot_general(
        x, w_ref[...], (((1,), (1,)), ((), ())),
        preferred_element_type=jnp.float32) + b_ref[...]


_mm_mid = pl.pallas_call(
    _mm_mid_body,
    out_shape=jax.ShapeDtypeStruct((_N, _D), jnp.float32),
)


def _sum_body(p_ref, o_ref):
    o_ref[...] = p_ref[0, :_N] + p_ref[1, :_N]


_sum_partials = pl.pallas_call(
    _sum_body,
    out_shape=jax.ShapeDtypeStruct((_N, _D), jnp.float32),
)


def kernel(t, y, edge_index, W1, b1, W2, b2, W3, b3):
    pad_src = jnp.zeros((_E_PAD - _E,), jnp.int32)
    pad_dst = _N + jnp.arange(_E_PAD - _E, dtype=jnp.int32) % (_NACC - _N)
    src = jnp.concatenate([edge_index[0], pad_src])
    dst = jnp.concatenate([edge_index[1], pad_dst])
    zeros = jnp.zeros((_RPT, _D), jnp.float32)

    z = _mm_first(y, W1, b1.reshape(1, _D))
    p = _spmm(z, src, dst, zeros)
    z = _mm_mid(p, W2, b2.reshape(1, _D))
    p = _spmm(z, src, dst, zeros)
    z = _mm_mid(p, W3, b3.reshape(1, _D))
    p = _spmm(z, src, dst, zeros)
    return _sum_partials(p)


# speedup vs baseline: 1.1875x; 1.1875x over previous
import jax
import jax.numpy as jnp
from jax import lax
from jax.experimental import pallas as pl
from jax.experimental.pallas import tpu as pltpu
from jax.experimental.pallas import tpu_sc as plsc

_N = 10000
_E = 320000
_D = 128
_CHUNK = 64
_NC = 2
_NS = 16
_NW = _NC * _NS
_E_PAD = 327680
_EPW = _E_PAD // _NW
_NACC = 10112
_RPT = _NACC // _NS


_EPH = _EPW // 2
_CPH = _EPH // _CHUNK
_G = 4
_DEPTH = 2
_NGRP = _CPH // _G


def _spmm_body(z_hbm, src_hbm, dst_hbm, zeros_hbm, out_hbm,
               src_v, dst_v, didx0, didx1, didx2, didx3,
               rows0, rows1, rows2, rows3, acc,
               gsem0, gsem1, gsem2, gsem3, ssem0, ssem1, ssem2, ssem3):
    rows = (rows0, rows1, rows2, rows3)
    didx = (didx0, didx1, didx2, didx3)
    gsem = (gsem0, gsem1, gsem2, gsem3)
    ssem = (ssem0, ssem1, ssem2, ssem3)
    c = lax.axis_index("c")
    s = lax.axis_index("s")
    wid = s * _NC + c

    def stage(h):
        pltpu.sync_copy(src_hbm.at[pl.ds(wid * _EPW + h * _EPH, _EPH)],
                        src_v)
        pltpu.sync_copy(dst_hbm.at[pl.ds(wid * _EPW + h * _EPH, _EPH)],
                        dst_v)

    pltpu.sync_copy(zeros_hbm, acc.at[pl.ds(s * _RPT, _RPT)])
    stage(0)
    plsc.subcore_barrier()

    def copy_didx(lch, b):
        for k in range(_CHUNK // 16):
            didx[b][pl.ds(k * 16, 16)] = dst_v[pl.ds(lch * _CHUNK + k * 16,
                                                     16)]

    def fire_gather(lch, b):
        pltpu.async_copy(z_hbm.at[src_v.at[pl.ds(lch * _CHUNK, _CHUNK)]],
                         rows[b], gsem[b])

    def wait_gather(lch, b):
        pltpu.make_async_copy(z_hbm.at[src_v.at[pl.ds(lch * _CHUNK,
                                                      _CHUNK)]],
                              rows[b], gsem[b]).wait()

    def fire_scatter(b):
        pltpu.async_copy(rows[b], acc.at[didx[b]], ssem[b], add=True)

    def wait_scatter(b):
        pltpu.make_async_copy(rows[b], acc.at[didx[b]], ssem[b]).wait()

    def run_half(first):
        for j in range(_G):
            if not first:
                wait_scatter(j)
            copy_didx(j, j)
            fire_gather(j, j)
            if j >= _DEPTH:
                wait_gather(j - _DEPTH, j - _DEPTH)
                fire_scatter(j - _DEPTH)

        @pl.loop(1, _NGRP)
        def _(g):
            for b in range(_G):
                k = g * _G + b
                wait_scatter(b)
                copy_didx(k, b)
                fire_gather(k, b)
                bd = (b + _G - _DEPTH) % _G
                wait_gather(k - _DEPTH, bd)
                fire_scatter(bd)

        for j in range(_CPH - _DEPTH, _CPH):
            bd = j % _G
            wait_gather(j, bd)
            fire_scatter(bd)

    run_half(True)
    stage(1)
    run_half(False)

    for b in range(_G):
        wait_scatter(b)

    plsc.subcore_barrier()
    pltpu.sync_copy(acc.at[pl.ds(s * _RPT, _RPT)],
                    out_hbm.at[c, pl.ds(s * _RPT, _RPT)])


_spmm = pl.kernel(
    _spmm_body,
    out_type=jax.ShapeDtypeStruct((_NC, _NACC, _D), jnp.float32),
    mesh=plsc.VectorSubcoreMesh(core_axis_name="c", subcore_axis_name="s",
                                num_cores=_NC, num_subcores=_NS),
    scratch_types=[
        pltpu.VMEM((_EPH,), jnp.int32),
        pltpu.VMEM((_EPH,), jnp.int32),
        pltpu.VMEM((_CHUNK,), jnp.int32),
        pltpu.VMEM((_CHUNK,), jnp.int32),
        pltpu.VMEM((_CHUNK,), jnp.int32),
        pltpu.VMEM((_CHUNK,), jnp.int32),
        pltpu.VMEM((_CHUNK, _D), jnp.float32),
        pltpu.VMEM((_CHUNK, _D), jnp.float32),
        pltpu.VMEM((_CHUNK, _D), jnp.float32),
        pltpu.VMEM((_CHUNK, _D), jnp.float32),
        pltpu.VMEM_SHARED((_NACC, _D), jnp.float32),
    ] + [pltpu.SemaphoreType.DMA] * (2 * _G),
)


def _mm_first_body(y_ref, w_ref, b_ref, o_ref):
    o_ref[...] = lax.dot_general(
        y_ref[...], w_ref[...], (((1,), (1,)), ((), ())),
        preferred_element_type=jnp.float32) + b_ref[...]


_mm_first = pl.pallas_call(
    _mm_first_body,
    out_shape=jax.ShapeDtypeStruct((_N, _D), jnp.float32),
)


def _mm_mid_body(p_ref, w_ref, b_ref, o_ref):
    x = jnp.maximum(p_ref[0, :_N] + p_ref[1, :_N], 0.0)
    o_ref[...] = lax.dot_general(
        x, w_ref[...], (((1,), (1,)), ((), ())),
        preferred_element_type=jnp.float32) + b_ref[...]


_mm_mid = pl.pallas_call(
    _mm_mid_body,
    out_shape=jax.ShapeDtypeStruct((_N, _D), jnp.float32),
)


def _sum_body(p_ref, o_ref):
    o_ref[...] = p_ref[0, :_N] + p_ref[1, :_N]


_sum_partials = pl.pallas_call(
    _sum_body,
    out_shape=jax.ShapeDtypeStruct((_N, _D), jnp.float32),
)


def kernel(t, y, edge_index, W1, b1, W2, b2, W3, b3):
    epw_real = _E // _NW
    npad = _EPW - epw_real
    pad_s = jnp.zeros((_NW, npad), jnp.int32)
    pad_d = jnp.broadcast_to(
        _N + jnp.arange(npad, dtype=jnp.int32) % (_NACC - _N), (_NW, npad))
    src = jnp.concatenate(
        [edge_index[0].reshape(_NW, epw_real), pad_s], axis=1).reshape(-1)
    dst = jnp.concatenate(
        [edge_index[1].reshape(_NW, epw_real), pad_d], axis=1).reshape(-1)
    zeros = jnp.zeros((_RPT, _D), jnp.float32)

    z = _mm_first(y, W1, b1.reshape(1, _D))
    p = _spmm(z, src, dst, zeros)
    z = _mm_mid(p, W2, b2.reshape(1, _D))
    p = _spmm(z, src, dst, zeros)
    z = _mm_mid(p, W3, b3.reshape(1, _D))
    p = _spmm(z, src, dst, zeros)
    return _sum_partials(p)


# speedup vs baseline: 3.7981x; 3.1985x over previous
import jax
import jax.numpy as jnp
from jax import lax
from jax.experimental import pallas as pl
from jax.experimental.pallas import tpu as pltpu
from jax.experimental.pallas import tpu_sc as plsc

_N = 10000
_E = 320000
_D = 128
_CHUNK = 64
_NC = 2
_NS = 16
_NW = _NC * _NS
_E_PAD = 327680
_EPW = _E_PAD // _NW
_NACC = 10112
_RPT = _NACC // _NS


_EPH = _EPW // 2
_CPH = _EPH // _CHUNK
_G = 4
_DEPTH = 2
_NGRP = _CPH // _G


def _spmm_body(z_hbm, src_hbm, dst_hbm, zeros_hbm, out_hbm,
               src_v, dst_v, didx0, didx1, didx2, didx3,
               rows0, rows1, rows2, rows3, acc,
               gsem0, gsem1, gsem2, gsem3, ssem0, ssem1, ssem2, ssem3):
    rows = (rows0, rows1, rows2, rows3)
    didx = (didx0, didx1, didx2, didx3)
    gsem = (gsem0, gsem1, gsem2, gsem3)
    ssem = (ssem0, ssem1, ssem2, ssem3)
    c = lax.axis_index("c")
    s = lax.axis_index("s")
    wid = s * _NC + c

    def stage(h):
        pltpu.sync_copy(src_hbm.at[pl.ds(wid * _EPW + h * _EPH, _EPH)],
                        src_v)
        pltpu.sync_copy(dst_hbm.at[pl.ds(wid * _EPW + h * _EPH, _EPH)],
                        dst_v)

    pltpu.sync_copy(zeros_hbm, acc.at[pl.ds(s * _RPT, _RPT)])
    stage(0)
    plsc.subcore_barrier()

    def copy_didx(lch, b):
        for k in range(_CHUNK // 16):
            didx[b][pl.ds(k * 16, 16)] = dst_v[pl.ds(lch * _CHUNK + k * 16,
                                                     16)]

    def fire_gather(lch, b):
        pltpu.async_copy(z_hbm.at[src_v.at[pl.ds(lch * _CHUNK, _CHUNK)]],
                         rows[b], gsem[b])

    def wait_gather(lch, b):
        pltpu.make_async_copy(z_hbm.at[src_v.at[pl.ds(lch * _CHUNK,
                                                      _CHUNK)]],
                              rows[b], gsem[b]).wait()

    def fire_scatter(b):
        pltpu.async_copy(rows[b], acc.at[didx[b]], ssem[b], add=True)

    def wait_scatter(b):
        pltpu.make_async_copy(rows[b], acc.at[didx[b]], ssem[b]).wait()

    def run_half(first):
        for j in range(_G):
            if not first:
                wait_scatter(j)
            copy_didx(j, j)
            fire_gather(j, j)
            if j >= _DEPTH:
                wait_gather(j - _DEPTH, j - _DEPTH)
                fire_scatter(j - _DEPTH)

        @pl.loop(1, _NGRP)
        def _(g):
            for b in range(_G):
                k = g * _G + b
                wait_scatter(b)
                copy_didx(k, b)
                fire_gather(k, b)
                bd = (b + _G - _DEPTH) % _G
                wait_gather(k - _DEPTH, bd)
                fire_scatter(bd)

        for j in range(_CPH - _DEPTH, _CPH):
            bd = j % _G
            wait_gather(j, bd)
            fire_scatter(bd)

    run_half(True)
    stage(1)
    run_half(False)

    for b in range(_G):
        wait_scatter(b)

    plsc.subcore_barrier()
    pltpu.sync_copy(acc.at[pl.ds(s * _RPT, _RPT)],
                    out_hbm.at[c, pl.ds(s * _RPT, _RPT)])


_spmm = pl.kernel(
    _spmm_body,
    out_type=jax.ShapeDtypeStruct((_NC, _NACC, _D), jnp.float32),
    mesh=plsc.VectorSubcoreMesh(core_axis_name="c", subcore_axis_name="s",
                                num_cores=_NC, num_subcores=_NS),
    scratch_types=[
        pltpu.VMEM((_EPH,), jnp.int32),
        pltpu.VMEM((_EPH,), jnp.int32),
        pltpu.VMEM((_CHUNK,), jnp.int32),
        pltpu.VMEM((_CHUNK,), jnp.int32),
        pltpu.VMEM((_CHUNK,), jnp.int32),
        pltpu.VMEM((_CHUNK,), jnp.int32),
        pltpu.VMEM((_CHUNK, _D), jnp.float32),
        pltpu.VMEM((_CHUNK, _D), jnp.float32),
        pltpu.VMEM((_CHUNK, _D), jnp.float32),
        pltpu.VMEM((_CHUNK, _D), jnp.float32),
        pltpu.VMEM_SHARED((_NACC, _D), jnp.float32),
    ] + [pltpu.SemaphoreType.DMA] * (2 * _G),
)


def _mm_first_body(y_ref, w_ref, b_ref, o_ref):
    o_ref[...] = lax.dot_general(
        y_ref[...], w_ref[...], (((1,), (1,)), ((), ())),
        preferred_element_type=jnp.float32) + b_ref[...]


_mm_first = pl.pallas_call(
    _mm_first_body,
    out_shape=jax.ShapeDtypeStruct((_N, _D), jnp.float32),
)


def _mm_mid_body(p_ref, w_ref, b_ref, o_ref):
    x = jnp.maximum(p_ref[0, :_N] + p_ref[1, :_N], 0.0)
    o_ref[...] = lax.dot_general(
        x, w_ref[...], (((1,), (1,)), ((), ())),
        preferred_element_type=jnp.float32) + b_ref[...]


_mm_mid = pl.pallas_call(
    _mm_mid_body,
    out_shape=jax.ShapeDtypeStruct((_N, _D), jnp.float32),
)


def _sum_body(p_ref, o_ref):
    o_ref[...] = p_ref[0, :_N] + p_ref[1, :_N]


_sum_partials = pl.pallas_call(
    _sum_body,
    out_shape=jax.ShapeDtypeStruct((_N, _D), jnp.float32),
)


def kernel(t, y, edge_index, W1, b1, W2, b2, W3, b3):
    epw_real = _E // _NW
    npad = _EPW - epw_real
    w = jnp.arange(_NW, dtype=jnp.int32)[:, None]
    i = jnp.arange(npad, dtype=jnp.int32)[None, :]
    pad_s = (w * 313 + i * 41) % _N
    pad_d = _N + (w % _NS) * 7 + i % 7
    src = jnp.concatenate(
        [edge_index[0].reshape(_NW, epw_real), pad_s], axis=1).reshape(-1)
    dst = jnp.concatenate(
        [edge_index[1].reshape(_NW, epw_real), pad_d], axis=1).reshape(-1)
    zeros = jnp.zeros((_RPT, _D), jnp.float32)

    z = _mm_first(y, W1, b1.reshape(1, _D))
    p = _spmm(z, src, dst, zeros)
    z = _mm_mid(p, W2, b2.reshape(1, _D))
    p = _spmm(z, src, dst, zeros)
    z = _mm_mid(p, W3, b3.reshape(1, _D))
    p = _spmm(z, src, dst, zeros)
    return _sum_partials(p)


# speedup vs baseline: 4.1144x; 1.0833x over previous
import jax
import jax.numpy as jnp
from jax import lax
from jax.experimental import pallas as pl
from jax.experimental.pallas import tpu as pltpu
from jax.experimental.pallas import tpu_sc as plsc

_N = 10000
_E = 320000
_D = 128
_CHUNK = 64
_NC = 2
_NS = 16
_NW = _NC * _NS
_E_PAD = 327680
_EPW = _E_PAD // _NW
_NACC = 10112
_RPT = _NACC // _NS


_EPH = _EPW // 2
_CPH = _EPH // _CHUNK
_G = 4
_DEPTH = 3
_NGRP = _CPH // _G


def _spmm_body(z_hbm, src_hbm, dst_hbm, zeros_hbm, out_hbm,
               src_v, dst_v, didx0, didx1, didx2, didx3,
               rows0, rows1, rows2, rows3, acc,
               gsem0, gsem1, gsem2, gsem3, ssem0, ssem1, ssem2, ssem3):
    rows = (rows0, rows1, rows2, rows3)
    didx = (didx0, didx1, didx2, didx3)
    gsem = (gsem0, gsem1, gsem2, gsem3)
    ssem = (ssem0, ssem1, ssem2, ssem3)
    c = lax.axis_index("c")
    s = lax.axis_index("s")
    wid = s * _NC + c

    def stage(h):
        pltpu.sync_copy(src_hbm.at[pl.ds(wid * _EPW + h * _EPH, _EPH)],
                        src_v)
        pltpu.sync_copy(dst_hbm.at[pl.ds(wid * _EPW + h * _EPH, _EPH)],
                        dst_v)

    pltpu.sync_copy(zeros_hbm, acc.at[pl.ds(s * _RPT, _RPT)])
    stage(0)
    plsc.subcore_barrier()

    def copy_didx(lch, b):
        for k in range(_CHUNK // 16):
            didx[b][pl.ds(k * 16, 16)] = dst_v[pl.ds(lch * _CHUNK + k * 16,
                                                     16)]

    def fire_gather(lch, b):
        pltpu.async_copy(z_hbm.at[src_v.at[pl.ds(lch * _CHUNK, _CHUNK)]],
                         rows[b], gsem[b])

    def wait_gather(lch, b):
        pltpu.make_async_copy(z_hbm.at[src_v.at[pl.ds(lch * _CHUNK,
                                                      _CHUNK)]],
                              rows[b], gsem[b]).wait()

    def fire_scatter(b):
        pltpu.async_copy(rows[b], acc.at[didx[b]], ssem[b], add=True)

    def wait_scatter(b):
        pltpu.make_async_copy(rows[b], acc.at[didx[b]], ssem[b]).wait()

    def run_half(first):
        for j in range(_G):
            if not first:
                wait_scatter(j)
            copy_didx(j, j)
            fire_gather(j, j)
            if j >= _DEPTH:
                wait_gather(j - _DEPTH, j - _DEPTH)
                fire_scatter(j - _DEPTH)

        @pl.loop(1, _NGRP)
        def _(g):
            for b in range(_G):
                k = g * _G + b
                wait_scatter(b)
                copy_didx(k, b)
                fire_gather(k, b)
                bd = (b + _G - _DEPTH) % _G
                wait_gather(k - _DEPTH, bd)
                fire_scatter(bd)

        for j in range(_CPH - _DEPTH, _CPH):
            bd = j % _G
            wait_gather(j, bd)
            fire_scatter(bd)

    run_half(True)
    stage(1)
    run_half(False)

    for b in range(_G):
        wait_scatter(b)

    plsc.subcore_barrier()
    pltpu.sync_copy(acc.at[pl.ds(s * _RPT, _RPT)],
                    out_hbm.at[c, pl.ds(s * _RPT, _RPT)])


_spmm = pl.kernel(
    _spmm_body,
    out_type=jax.ShapeDtypeStruct((_NC, _NACC, _D), jnp.float32),
    mesh=plsc.VectorSubcoreMesh(core_axis_name="c", subcore_axis_name="s",
                                num_cores=_NC, num_subcores=_NS),
    scratch_types=[
        pltpu.VMEM((_EPH,), jnp.int32),
        pltpu.VMEM((_EPH,), jnp.int32),
        pltpu.VMEM((_CHUNK,), jnp.int32),
        pltpu.VMEM((_CHUNK,), jnp.int32),
        pltpu.VMEM((_CHUNK,), jnp.int32),
        pltpu.VMEM((_CHUNK,), jnp.int32),
        pltpu.VMEM((_CHUNK, _D), jnp.float32),
        pltpu.VMEM((_CHUNK, _D), jnp.float32),
        pltpu.VMEM((_CHUNK, _D), jnp.float32),
        pltpu.VMEM((_CHUNK, _D), jnp.float32),
        pltpu.VMEM_SHARED((_NACC, _D), jnp.float32),
    ] + [pltpu.SemaphoreType.DMA] * (2 * _G),
)


def _mm_first_body(y_ref, w_ref, b_ref, o_ref):
    o_ref[...] = lax.dot_general(
        y_ref[...], w_ref[...], (((1,), (1,)), ((), ())),
        preferred_element_type=jnp.float32) + b_ref[...]


_mm_first = pl.pallas_call(
    _mm_first_body,
    out_shape=jax.ShapeDtypeStruct((_N, _D), jnp.float32),
)


def _mm_mid_body(p_ref, w_ref, b_ref, o_ref):
    x = jnp.maximum(p_ref[0, :_N] + p_ref[1, :_N], 0.0)
    o_ref[...] = lax.dot_general(
        x, w_ref[...], (((1,), (1,)), ((), ())),
        preferred_element_type=jnp.float32) + b_ref[...]


_mm_mid = pl.pallas_call(
    _mm_mid_body,
    out_shape=jax.ShapeDtypeStruct((_N, _D), jnp.float32),
)


def _sum_body(p_ref, o_ref):
    o_ref[...] = p_ref[0, :_N] + p_ref[1, :_N]


_sum_partials = pl.pallas_call(
    _sum_body,
    out_shape=jax.ShapeDtypeStruct((_N, _D), jnp.float32),
)


def kernel(t, y, edge_index, W1, b1, W2, b2, W3, b3):
    epw_real = _E // _NW
    npad = _EPW - epw_real
    w = jnp.arange(_NW, dtype=jnp.int32)[:, None]
    i = jnp.arange(npad, dtype=jnp.int32)[None, :]
    pad_s = (w * 313 + i * 41) % _N
    pad_d = _N + (w % _NS) * 7 + i % 7
    src = jnp.concatenate(
        [edge_index[0].reshape(_NW, epw_real), pad_s], axis=1).reshape(-1)
    dst = jnp.concatenate(
        [edge_index[1].reshape(_NW, epw_real), pad_d], axis=1).reshape(-1)
    zeros = jnp.zeros((_RPT, _D), jnp.float32)

    z = _mm_first(y, W1, b1.reshape(1, _D))
    p = _spmm(z, src, dst, zeros)
    z = _mm_mid(p, W2, b2.reshape(1, _D))
    p = _spmm(z, src, dst, zeros)
    z = _mm_mid(p, W3, b3.reshape(1, _D))
    p = _spmm(z, src, dst, zeros)
    return _sum_partials(p)
